# Initial kernel scaffold; baseline (speedup 1.0000x reference)
import jax
import jax.numpy as jnp
from jax.experimental import pallas as pl


def kernel(token_ids, embedding_table):
    raise NotImplementedError("write your pallas kernel here")


# speedup vs baseline: 1.8436x; 1.8436x over previous
import functools

import jax
import jax.numpy as jnp
from jax import lax
from jax.experimental import pallas as pl
from jax.experimental.pallas import tpu as pltpu
from jax.experimental.pallas import tpu_sc as plsc


@functools.cache
def _make_gather(B, V, D, NW, NC, K):
    G = 128
    C = K * G
    b_per_w = B // NW
    nchunk = b_per_w // C
    mesh = plsc.VectorSubcoreMesh(core_axis_name="c", subcore_axis_name="s")

    @functools.partial(
        pl.kernel,
        mesh=mesh,
        out_type=jax.ShapeDtypeStruct((B, D), jnp.float32),
        scratch_types=[
            pltpu.VMEM((K, G), jnp.int32),
            pltpu.VMEM((C, D), jnp.float32),
            pltpu.SemaphoreType.DMA,
        ],
        compiler_params=pltpu.CompilerParams(use_tc_tiling_on_sc=False),
    )
    def gather_kernel(idx_hbm, table_hbm, out_hbm, idx_v, rows_v, sem):
        wid = lax.axis_index("s") * NC + lax.axis_index("c")
        base = wid * b_per_w

        def body(i, carry):
            pltpu.sync_copy(idx_hbm.at[wid, i], idx_v)
            copies = [
                pltpu.async_copy(
                    table_hbm.at[idx_v.at[j]],
                    rows_v.at[pl.ds(j * G, G)],
                    sem,
                )
                for j in range(K)
            ]
            for cp in copies:
                cp.wait()
            pltpu.sync_copy(rows_v, out_hbm.at[pl.ds(base + i * C, C)])
            return carry

        lax.fori_loop(0, nchunk, body, 0)

    return gather_kernel


def kernel(token_ids, embedding_table):
    Bt, H = token_ids.shape
    V, D = embedding_table.shape
    B = Bt * H
    NW, NC = 32, 2
    K = 8
    idx = token_ids.reshape(NW, -1, K, 128).astype(jnp.int32)
    out = _make_gather(B, V, D, NW, NC, K)(idx, embedding_table)
    return out.reshape(Bt, H, D)


# speedup vs baseline: 1.8718x; 1.0153x over previous
import functools

import jax
import jax.numpy as jnp
from jax import lax
from jax.experimental import pallas as pl
from jax.experimental.pallas import tpu as pltpu
from jax.experimental.pallas import tpu_sc as plsc


@functools.cache
def _make_gather(B, V, D, NW, NC, C, NBUF):
    b_per_w = B // NW
    nchunk = b_per_w // C
    assert nchunk % NBUF == 0
    mesh = plsc.VectorSubcoreMesh(core_axis_name="c", subcore_axis_name="s")

    @functools.partial(
        pl.kernel,
        mesh=mesh,
        out_type=jax.ShapeDtypeStruct((B, D), jnp.float32),
        scratch_types=[
            pltpu.VMEM((nchunk, C), jnp.int32),
            [pltpu.VMEM((C, D), jnp.float32) for _ in range(NBUF)],
            [pltpu.SemaphoreType.DMA for _ in range(NBUF)],
        ],
        compiler_params=pltpu.CompilerParams(use_tc_tiling_on_sc=False),
    )
    def gather_kernel(idx_hbm, table_hbm, out_hbm, idx_v, bufs, sems):
        wid = lax.axis_index("s") * NC + lax.axis_index("c")
        base = wid * b_per_w
        pltpu.sync_copy(idx_hbm.at[wid], idx_v)

        def fire(g, b):
            pltpu.async_copy(table_hbm.at[idx_v.at[g]], bufs[b], sems[b])

        def drain(g, b):
            pltpu.make_async_copy(table_hbm.at[idx_v.at[g]], bufs[b], sems[b]).wait()

        fire(0, 0)

        def body(i, carry):
            for b in range(NBUF):
                g = i * NBUF + b
                nb = (b + 1) % NBUF

                @pl.when(g + 1 < nchunk)
                def _():
                    fire(g + 1, nb)

                drain(g, b)
                pltpu.sync_copy(bufs[b], out_hbm.at[pl.ds(base + g * C, C)])
            return carry

        lax.fori_loop(0, nchunk // NBUF, body, 0)

    return gather_kernel


def kernel(token_ids, embedding_table):
    Bt, H = token_ids.shape
    V, D = embedding_table.shape
    B = Bt * H
    NW, NC = 32, 2
    C, NBUF = 640, 2
    idx = token_ids.reshape(NW, (B // NW) // C, C).astype(jnp.int32)
    out = _make_gather(B, V, D, NW, NC, C, NBUF)(idx, embedding_table)
    return out.reshape(Bt, H, D)


# speedup vs baseline: 1.8747x; 1.0015x over previous
import functools

import jax
import jax.numpy as jnp
from jax import lax
from jax.experimental import pallas as pl
from jax.experimental.pallas import tpu as pltpu
from jax.experimental.pallas import tpu_sc as plsc


@functools.cache
def _make_gather(Bt, H, B, V, D, NW, NC, C, NBUF):
    b_per_w = B // NW
    nchunk = b_per_w // C
    assert nchunk % NBUF == 0
    mesh = plsc.VectorSubcoreMesh(core_axis_name="c", subcore_axis_name="s")

    @functools.partial(
        pl.kernel,
        mesh=mesh,
        out_type=jax.ShapeDtypeStruct((Bt, H, D), jnp.float32),
        scratch_types=[
            pltpu.VMEM((nchunk, C), jnp.int32),
            [pltpu.VMEM((C, D), jnp.float32) for _ in range(NBUF)],
            [pltpu.SemaphoreType.DMA for _ in range(NBUF)],
        ],
        compiler_params=pltpu.CompilerParams(use_tc_tiling_on_sc=False),
    )
    def gather_kernel(idx_hbm, table_hbm, out_hbm, idx_v, bufs, sems):
        wid = lax.axis_index("s") * NC + lax.axis_index("c")
        base = wid * b_per_w
        pltpu.sync_copy(idx_hbm.at[wid], idx_v)

        def fire(g, b):
            pltpu.async_copy(table_hbm.at[idx_v.at[g]], bufs[b], sems[b])

        def drain(g, b):
            pltpu.make_async_copy(table_hbm.at[idx_v.at[g]], bufs[b], sems[b]).wait()

        fire(0, 0)

        KB = C // H

        def body(i, carry):
            for b in range(NBUF):
                g = i * NBUF + b
                nb = (b + 1) % NBUF

                @pl.when(g + 1 < nchunk)
                def _():
                    fire(g + 1, nb)

                drain(g, b)
                b0 = (base + g * C) // H
                for kb in range(KB):
                    pltpu.sync_copy(
                        bufs[b].at[pl.ds(kb * H, H)], out_hbm.at[b0 + kb]
                    )
            return carry

        lax.fori_loop(0, nchunk // NBUF, body, 0)

    return gather_kernel


def kernel(token_ids, embedding_table):
    Bt, H = token_ids.shape
    V, D = embedding_table.shape
    B = Bt * H
    NW, NC = 32, 2
    C, NBUF = 8 * H, 2
    idx = token_ids.reshape(NW, (B // NW) // C, C).astype(jnp.int32)
    return _make_gather(Bt, H, B, V, D, NW, NC, C, NBUF)(idx, embedding_table)
